# Initial kernel scaffold; baseline (speedup 1.0000x reference)
import jax
import jax.numpy as jnp
from jax.experimental import pallas as pl


def kernel(nodes, edges, edge_attr, params):
    raise NotImplementedError("write your pallas kernel here")


# speedup vs baseline: 4.4506x; 4.4506x over previous
import functools

import jax
import jax.numpy as jnp
from jax import lax
from jax.experimental import pallas as pl
from jax.experimental.pallas import tpu as pltpu
from jax.experimental.pallas import tpu_sc as plsc

N = 2048
E = 65536
H = 128
EMB = 32
NW = 32
EPW = E // NW
CHUNK = 128
NCHUNK = EPW // CHUNK

_MESH = plsc.VectorSubcoreMesh(core_axis_name="c", subcore_axis_name="s")
_PREC = lax.Precision.HIGHEST


def _silu(x):
    return x * jax.nn.sigmoid(x)


@functools.partial(
    pl.kernel,
    out_type=jax.ShapeDtypeStruct((E, H), jnp.float32),
    mesh=_MESH,
    scratch_types=[
        pltpu.VMEM((NCHUNK, CHUNK), jnp.int32),
        pltpu.VMEM((CHUNK, H), jnp.float32),
        pltpu.SemaphoreType.DMA,
    ],
)
def _sc_gather(table_hbm, idx_hbm, out_hbm, idx_v, rows_v, sem):
    w = lax.axis_index("s") * 2 + lax.axis_index("c")
    pltpu.sync_copy(idx_hbm.at[pl.ds(w * NCHUNK, NCHUNK), :], idx_v)

    def chunk(j, carry):
        pltpu.async_copy(table_hbm.at[idx_v.at[j]], rows_v, sem).wait()
        pltpu.sync_copy(rows_v, out_hbm.at[pl.ds(w * EPW + j * CHUNK, CHUNK), :])
        return carry

    lax.fori_loop(0, NCHUNK, chunk, 0)


@functools.partial(
    pl.kernel,
    out_type=jax.ShapeDtypeStruct((2 * N, H), jnp.float32),
    mesh=_MESH,
    scratch_types=[
        pltpu.VMEM((NCHUNK, CHUNK), jnp.int32),
        pltpu.VMEM((CHUNK, H), jnp.float32),
        pltpu.VMEM_SHARED((N, H), jnp.float32),
    ],
)
def _sc_scatter(m_hbm, idx_hbm, out_hbm, idx_v, rows_v, acc_shared):
    c = lax.axis_index("c")
    s = lax.axis_index("s")
    w = s * 2 + c
    rpw = N // 16

    def zrow(r, carry):
        for l in range(H // 16):
            rows_v[r, pl.ds(l * 16, 16)] = jnp.zeros((16,), jnp.float32)
        return carry

    lax.fori_loop(0, CHUNK, zrow, 0)
    pltpu.sync_copy(rows_v, acc_shared.at[pl.ds(s * rpw, rpw), :])
    plsc.subcore_barrier()

    pltpu.sync_copy(idx_hbm.at[pl.ds(w * NCHUNK, NCHUNK), :], idx_v)

    def chunk(j, carry):
        pltpu.sync_copy(m_hbm.at[pl.ds(w * EPW + j * CHUNK, CHUNK), :], rows_v)
        pltpu.sync_copy(rows_v, acc_shared.at[idx_v.at[j]], add=True)
        return carry

    lax.fori_loop(0, NCHUNK, chunk, 0)
    plsc.subcore_barrier()
    pltpu.sync_copy(acc_shared.at[pl.ds(s * rpw, rpw), :],
                    out_hbm.at[pl.ds(c * N + s * rpw, rpw), :])


BLK_E = 4096


def _edge_body(tA, tB, attr, w0c, W1T, b1, out):
    t = tA[...] + tB[...] + attr[...] * w0c[...]
    u = t * jax.nn.sigmoid(t)
    v = jnp.dot(u, W1T[...], preferred_element_type=jnp.float32,
                precision=_PREC) + b1[...]
    out[...] = v * jax.nn.sigmoid(v)


def _tc_edge(tA, tB, attr, w0c, W1T, b1):
    return pl.pallas_call(
        _edge_body,
        grid=(E // BLK_E,),
        in_specs=[
            pl.BlockSpec((BLK_E, H), lambda i: (i, 0)),
            pl.BlockSpec((BLK_E, H), lambda i: (i, 0)),
            pl.BlockSpec((BLK_E, 1), lambda i: (i, 0)),
            pl.BlockSpec((1, H), lambda i: (0, 0)),
            pl.BlockSpec((H, H), lambda i: (0, 0)),
            pl.BlockSpec((1, H), lambda i: (0, 0)),
        ],
        out_specs=pl.BlockSpec((BLK_E, H), lambda i: (i, 0)),
        out_shape=jax.ShapeDtypeStruct((E, H), jnp.float32),
    )(tA, tB, attr, w0c, W1T, b1)


def _node_body(h, agg2, WhT, WaT, bn0, Wn1T, bn1, WaTn, b0n, WbTn,
               h_out, A_out, B_out, *, first, last):
    g = agg2[...]
    agg = g[0:N] + g[N:2 * N]
    hh = h[...]
    if first:
        pre = hh * WhT[...] + bn0[...]
    else:
        pre = jnp.dot(hh, WhT[...], preferred_element_type=jnp.float32,
                      precision=_PREC) + bn0[...]
    pre = pre + jnp.dot(agg, WaT[...], preferred_element_type=jnp.float32,
                        precision=_PREC)
    u = pre * jax.nn.sigmoid(pre)
    hn = jnp.dot(u, Wn1T[...], preferred_element_type=jnp.float32,
                 precision=_PREC) + bn1[...]
    if not first:
        hn = hn + hh
    h_out[...] = hn
    if not last:
        A_out[...] = jnp.dot(hn, WaTn[...], preferred_element_type=jnp.float32,
                             precision=_PREC) + b0n[...]
        B_out[...] = jnp.dot(hn, WbTn[...], preferred_element_type=jnp.float32,
                             precision=_PREC)


def _tc_node(h, agg2, WhT, WaT, bn0, Wn1T, bn1, WaTn, b0n, WbTn,
             first, last):
    fin = 1 if first else H
    full = lambda shp: pl.BlockSpec(shp, lambda: tuple(0 for _ in shp))
    if last:
        body = functools.partial(_node_body, first=first, last=True)

        def body_last(h, agg2, WhT, WaT, bn0, Wn1T, bn1, h_out):
            body(h, agg2, WhT, WaT, bn0, Wn1T, bn1, None, None, None,
                 h_out, None, None)

        return pl.pallas_call(
            body_last,
            in_specs=[full((N, fin)), full((2 * N, H)),
                      full((fin, H)) if not first else full((1, H)),
                      full((H, H)), full((1, H)), full((H, H)), full((1, H))],
            out_specs=full((N, H)),
            out_shape=jax.ShapeDtypeStruct((N, H), jnp.float32),
        )(h, agg2, WhT, WaT, bn0, Wn1T, bn1)
    body = functools.partial(_node_body, first=first, last=False)
    return pl.pallas_call(
        body,
        in_specs=[full((N, fin)), full((2 * N, H)),
                  full((fin, H)) if not first else full((1, H)),
                  full((H, H)), full((1, H)), full((H, H)), full((1, H)),
                  full((H, H)), full((1, H)), full((H, H))],
        out_specs=[full((N, H))] * 3,
        out_shape=[jax.ShapeDtypeStruct((N, H), jnp.float32)] * 3,
    )(h, agg2, WhT, WaT, bn0, Wn1T, bn1, WaTn, b0n, WbTn)


def _prep0_body(noise, wa, b0, wb, A_out, B_out):
    nz = noise[...]
    A_out[...] = nz * wa[...] + b0[...]
    B_out[...] = nz * wb[...]


def _tc_prep0(noise, wa, b0, wb):
    full = lambda shp: pl.BlockSpec(shp, lambda: tuple(0 for _ in shp))
    return pl.pallas_call(
        _prep0_body,
        in_specs=[full((N, 1)), full((1, H)), full((1, H)), full((1, H))],
        out_specs=[full((N, H))] * 2,
        out_shape=[jax.ShapeDtypeStruct((N, H), jnp.float32)] * 2,
    )(noise, wa, b0, wb)


BLK_R = 256


def _dec_body(hb, hf, WeT, be, wd, bd, adj_out, x_out):
    i = pl.program_id(0)
    xb = jnp.dot(hb[...], WeT[...], preferred_element_type=jnp.float32,
                 precision=_PREC) + be[...]
    xf = jnp.dot(hf[...], WeT[...], preferred_element_type=jnp.float32,
                 precision=_PREC) + be[...]
    qb = xb * wd[...]
    sb = jnp.sum(qb * xb, axis=1, keepdims=True)
    srow = lax.dot_general(wd[...], xf * xf, (((1,), (1,)), ((), ())),
                           preferred_element_type=jnp.float32,
                           precision=_PREC)
    G = lax.dot_general(qb, xf, (((1,), (1,)), ((), ())),
                        preferred_element_type=jnp.float32, precision=_PREC)
    z = sb + srow - 2.0 * G + bd[...]
    a = jax.nn.sigmoid(z)
    rid = lax.broadcasted_iota(jnp.int32, (BLK_R, N), 0) + i * BLK_R
    cid = lax.broadcasted_iota(jnp.int32, (BLK_R, N), 1)
    adj_out[...] = jnp.where(rid == cid, 0.0, a)
    x_out[...] = xb


def _tc_decode(h, WeT, be, wd, bd):
    return pl.pallas_call(
        _dec_body,
        grid=(N // BLK_R,),
        in_specs=[
            pl.BlockSpec((BLK_R, H), lambda i: (i, 0)),
            pl.BlockSpec((N, H), lambda i: (0, 0)),
            pl.BlockSpec((H, EMB), lambda i: (0, 0)),
            pl.BlockSpec((1, EMB), lambda i: (0, 0)),
            pl.BlockSpec((1, EMB), lambda i: (0, 0)),
            pl.BlockSpec((1, 1), lambda i: (0, 0)),
        ],
        out_specs=[
            pl.BlockSpec((BLK_R, N), lambda i: (i, 0)),
            pl.BlockSpec((BLK_R, EMB), lambda i: (i, 0)),
        ],
        out_shape=[
            jax.ShapeDtypeStruct((N, N), jnp.float32),
            jax.ShapeDtypeStruct((N, EMB), jnp.float32),
        ],
    )(h, h, WeT, be, wd, bd)


def kernel(nodes, edges, edge_attr, params):
    del nodes
    row2d = edges[0].reshape(E // CHUNK, CHUNK)
    col2d = edges[1].reshape(E // CHUNK, CHUNK)
    noise = jax.random.normal(jax.random.key(1), (N, 1), dtype=jnp.float32)

    g0 = params["gcl_0"]["edge_mlp_0"]
    A, B = _tc_prep0(noise, g0["W"][:, 0:1].T, g0["b"].reshape(1, H),
                     g0["W"][:, 1:2].T)
    h = noise
    for i in range(4):
        g = params["gcl_%d" % i]
        fin = 1 if i == 0 else H
        W0 = g["edge_mlp_0"]["W"]
        w0c = W0[:, 2 * fin].reshape(1, H)
        W1T = g["edge_mlp_1"]["W"].T
        b1 = g["edge_mlp_1"]["b"].reshape(1, H)

        tA = _sc_gather(A, row2d)
        tB = _sc_gather(B, col2d)
        m = _tc_edge(tA, tB, edge_attr, w0c, W1T, b1)
        agg2 = _sc_scatter(m, row2d)

        Wn0 = g["node_mlp_0"]["W"]
        WhT = Wn0[:, :fin].T
        WaT = Wn0[:, fin:].T
        bn0 = g["node_mlp_0"]["b"].reshape(1, H)
        Wn1T = g["node_mlp_1"]["W"].T
        bn1 = g["node_mlp_1"]["b"].reshape(1, H)
        if i < 3:
            gn = params["gcl_%d" % (i + 1)]["edge_mlp_0"]
            h, A, B = _tc_node(h, agg2, WhT, WaT, bn0, Wn1T, bn1,
                               gn["W"][:, :H].T, gn["b"].reshape(1, H),
                               gn["W"][:, H:2 * H].T, first=(i == 0),
                               last=False)
        else:
            h = _tc_node(h, agg2, WhT, WaT, bn0, Wn1T, bn1,
                         None, None, None, first=False, last=True)

    fe, fd = params["fc_emb"], params["fc_dec"]
    adj, x = _tc_decode(h, fe["W"].T, fe["b"].reshape(1, EMB),
                        fd["W"].reshape(1, EMB), fd["b"].reshape(1, 1))
    return adj, x


# speedup vs baseline: 5.9641x; 1.3401x over previous
import functools

import jax
import jax.numpy as jnp
from jax import lax
from jax.experimental import pallas as pl
from jax.experimental.pallas import tpu as pltpu
from jax.experimental.pallas import tpu_sc as plsc

N = 2048
E = 65536
H = 128
EMB = 32
NW = 32
EPW = E // NW
CHUNK = 128
NCHUNK = EPW // CHUNK
NBUF = 3

_MESH = plsc.VectorSubcoreMesh(core_axis_name="c", subcore_axis_name="s")
_PREC = lax.Precision.HIGHEST


def _silu(x):
    return x * jax.nn.sigmoid(x)


_GATHER_SCRATCH = (
    [pltpu.VMEM((NCHUNK, CHUNK), jnp.int32)] * 2
    + [pltpu.VMEM((CHUNK, H), jnp.float32)] * (2 * NBUF)
    + [pltpu.SemaphoreType.DMA] * (3 * NBUF)
)


@functools.partial(
    pl.kernel,
    out_type=jax.ShapeDtypeStruct((E, H), jnp.float32),
    mesh=_MESH,
    scratch_types=_GATHER_SCRATCH,
)
def _sc_gather_add(A_hbm, B_hbm, row_hbm, col_hbm, out_hbm, ir, ic, *bufs):
    bA = bufs[0:NBUF]
    bB = bufs[NBUF:2 * NBUF]
    sA = bufs[2 * NBUF:2 * NBUF + NBUF]
    sB = bufs[3 * NBUF:3 * NBUF + NBUF]
    sS = bufs[4 * NBUF:4 * NBUF + NBUF]
    w = lax.axis_index("s") * 2 + lax.axis_index("c")
    pltpu.sync_copy(row_hbm.at[pl.ds(w * NCHUNK, NCHUNK), :], ir)
    pltpu.sync_copy(col_hbm.at[pl.ds(w * NCHUNK, NCHUNK), :], ic)

    def start_gather(j):
        p = j % NBUF
        return (pltpu.async_copy(A_hbm.at[ir.at[j]], bA[p], sA[p]),
                pltpu.async_copy(B_hbm.at[ic.at[j]], bB[p], sB[p]))

    inflight = {0: start_gather(0), 1: start_gather(1)}
    stores = {}
    for j in range(NCHUNK):
        p = j % NBUF
        ga, gb = inflight.pop(j)
        ga.wait()
        gb.wait()

        def addrow(r, carry, p=p):
            for l in range(H // 16):
                bA[p][r, pl.ds(l * 16, 16)] += bB[p][r, pl.ds(l * 16, 16)]
            return carry

        lax.fori_loop(0, CHUNK, addrow, 0)
        stores[j] = pltpu.async_copy(
            bA[p], out_hbm.at[pl.ds(w * EPW + j * CHUNK, CHUNK), :], sS[p])
        if j + 2 < NCHUNK:
            jn = j + 2
            if jn - NBUF in stores:
                stores.pop(jn - NBUF).wait()
            inflight[jn] = start_gather(jn)
    for j in sorted(stores):
        stores.pop(j).wait()


def _prep0_body(noise, wa, b0, wb, A_out, B_out):
    nz = noise[...]
    A_out[...] = nz * wa[...] + b0[...]
    B_out[...] = nz * wb[...]


def _tc_prep0(noise, wa, b0, wb):
    full = lambda shp: pl.BlockSpec(shp, lambda: tuple(0 for _ in shp))
    return pl.pallas_call(
        _prep0_body,
        in_specs=[full((N, 1)), full((1, H)), full((1, H)), full((1, H))],
        out_specs=[full((N, H))] * 2,
        out_shape=[jax.ShapeDtypeStruct((N, H), jnp.float32)] * 2,
    )(noise, wa, b0, wb)


@functools.partial(
    pl.kernel,
    out_type=jax.ShapeDtypeStruct((2 * N, H), jnp.float32),
    mesh=_MESH,
    scratch_types=[
        pltpu.VMEM((NCHUNK, CHUNK), jnp.int32),
        pltpu.VMEM((CHUNK, H), jnp.float32),
        pltpu.VMEM((CHUNK, H), jnp.float32),
        pltpu.VMEM_SHARED((N, H), jnp.float32),
        pltpu.SemaphoreType.DMA,
        pltpu.SemaphoreType.DMA,
    ],
)
def _sc_scatter(m_hbm, idx_hbm, out_hbm, idx_v, mb0, mb1, acc_shared, s0, s1):
    c = lax.axis_index("c")
    s = lax.axis_index("s")
    w = s * 2 + c
    mb = (mb0, mb1)
    sm = (s0, s1)
    rpw = N // 16

    def zrow(r, carry):
        for l in range(H // 16):
            mb0[r, pl.ds(l * 16, 16)] = jnp.zeros((16,), jnp.float32)
        return carry

    lax.fori_loop(0, CHUNK, zrow, 0)
    pltpu.sync_copy(mb0, acc_shared.at[pl.ds(s * rpw, rpw), :])
    plsc.subcore_barrier()

    pltpu.sync_copy(idx_hbm.at[pl.ds(w * NCHUNK, NCHUNK), :], idx_v)

    def load(j):
        p = j % 2
        return pltpu.async_copy(
            m_hbm.at[pl.ds(w * EPW + j * CHUNK, CHUNK), :], mb[p], sm[p])

    pend = {0: load(0)}
    for j in range(NCHUNK):
        p = j % 2
        pend.pop(j).wait()
        if j + 1 < NCHUNK:
            pend[j + 1] = load(j + 1)
        pltpu.sync_copy(mb[p], acc_shared.at[idx_v.at[j]], add=True)
    plsc.subcore_barrier()
    pltpu.sync_copy(acc_shared.at[pl.ds(s * rpw, rpw), :],
                    out_hbm.at[pl.ds(c * N + s * rpw, rpw), :])


BLK_E = 4096


def _edge_body(t, attr, w0c, W1T, b1, out):
    tt = t[...] + attr[...] * w0c[...]
    u = tt * jax.nn.sigmoid(tt)
    v = jnp.dot(u, W1T[...], preferred_element_type=jnp.float32,
                precision=_PREC) + b1[...]
    out[...] = v * jax.nn.sigmoid(v)


def _tc_edge(t, attr, w0c, W1T, b1):
    return pl.pallas_call(
        _edge_body,
        grid=(E // BLK_E,),
        in_specs=[
            pl.BlockSpec((BLK_E, H), lambda i: (i, 0)),
            pl.BlockSpec((BLK_E, 1), lambda i: (i, 0)),
            pl.BlockSpec((1, H), lambda i: (0, 0)),
            pl.BlockSpec((H, H), lambda i: (0, 0)),
            pl.BlockSpec((1, H), lambda i: (0, 0)),
        ],
        out_specs=pl.BlockSpec((BLK_E, H), lambda i: (i, 0)),
        out_shape=jax.ShapeDtypeStruct((E, H), jnp.float32),
    )(t, attr, w0c, W1T, b1)


def _node_body(h, agg2, WhT, WaT, bn0, Wn1T, bn1, WaTn, b0n, WbTn,
               h_out, A_out, B_out, *, first, last):
    g = agg2[...]
    agg = g[0:N] + g[N:2 * N]
    hh = h[...]
    if first:
        pre = hh * WhT[...] + bn0[...]
    else:
        pre = jnp.dot(hh, WhT[...], preferred_element_type=jnp.float32,
                      precision=_PREC) + bn0[...]
    pre = pre + jnp.dot(agg, WaT[...], preferred_element_type=jnp.float32,
                        precision=_PREC)
    u = pre * jax.nn.sigmoid(pre)
    hn = jnp.dot(u, Wn1T[...], preferred_element_type=jnp.float32,
                 precision=_PREC) + bn1[...]
    if not first:
        hn = hn + hh
    h_out[...] = hn
    if not last:
        A_out[...] = jnp.dot(hn, WaTn[...], preferred_element_type=jnp.float32,
                             precision=_PREC) + b0n[...]
        B_out[...] = jnp.dot(hn, WbTn[...], preferred_element_type=jnp.float32,
                             precision=_PREC)


def _tc_node(h, agg2, WhT, WaT, bn0, Wn1T, bn1, WaTn, b0n, WbTn,
             first, last):
    fin = 1 if first else H
    full = lambda shp: pl.BlockSpec(shp, lambda: tuple(0 for _ in shp))
    if last:
        body = functools.partial(_node_body, first=first, last=True)

        def body_last(h, agg2, WhT, WaT, bn0, Wn1T, bn1, h_out):
            body(h, agg2, WhT, WaT, bn0, Wn1T, bn1, None, None, None,
                 h_out, None, None)

        return pl.pallas_call(
            body_last,
            in_specs=[full((N, fin)), full((2 * N, H)),
                      full((fin, H)) if not first else full((1, H)),
                      full((H, H)), full((1, H)), full((H, H)), full((1, H))],
            out_specs=full((N, H)),
            out_shape=jax.ShapeDtypeStruct((N, H), jnp.float32),
        )(h, agg2, WhT, WaT, bn0, Wn1T, bn1)
    body = functools.partial(_node_body, first=first, last=False)
    return pl.pallas_call(
        body,
        in_specs=[full((N, fin)), full((2 * N, H)),
                  full((fin, H)) if not first else full((1, H)),
                  full((H, H)), full((1, H)), full((H, H)), full((1, H)),
                  full((H, H)), full((1, H)), full((H, H))],
        out_specs=[full((N, H))] * 3,
        out_shape=[jax.ShapeDtypeStruct((N, H), jnp.float32)] * 3,
    )(h, agg2, WhT, WaT, bn0, Wn1T, bn1, WaTn, b0n, WbTn)


BLK_R = 256


def _dec_body(hb, hf, WeT, be, wd, bd, adj_out, x_out):
    i = pl.program_id(0)
    xb = jnp.dot(hb[...], WeT[...], preferred_element_type=jnp.float32,
                 precision=_PREC) + be[...]
    xf = jnp.dot(hf[...], WeT[...], preferred_element_type=jnp.float32,
                 precision=_PREC) + be[...]
    qb = xb * wd[...]
    sb = jnp.sum(qb * xb, axis=1, keepdims=True)
    srow = lax.dot_general(wd[...], xf * xf, (((1,), (1,)), ((), ())),
                           preferred_element_type=jnp.float32,
                           precision=_PREC)
    G = lax.dot_general(qb, xf, (((1,), (1,)), ((), ())),
                        preferred_element_type=jnp.float32, precision=_PREC)
    z = sb + srow - 2.0 * G + bd[...]
    a = jax.nn.sigmoid(z)
    rid = lax.broadcasted_iota(jnp.int32, (BLK_R, N), 0) + i * BLK_R
    cid = lax.broadcasted_iota(jnp.int32, (BLK_R, N), 1)
    adj_out[...] = jnp.where(rid == cid, 0.0, a)
    x_out[...] = xb


def _tc_decode(h, WeT, be, wd, bd):
    return pl.pallas_call(
        _dec_body,
        grid=(N // BLK_R,),
        in_specs=[
            pl.BlockSpec((BLK_R, H), lambda i: (i, 0)),
            pl.BlockSpec((N, H), lambda i: (0, 0)),
            pl.BlockSpec((H, EMB), lambda i: (0, 0)),
            pl.BlockSpec((1, EMB), lambda i: (0, 0)),
            pl.BlockSpec((1, EMB), lambda i: (0, 0)),
            pl.BlockSpec((1, 1), lambda i: (0, 0)),
        ],
        out_specs=[
            pl.BlockSpec((BLK_R, N), lambda i: (i, 0)),
            pl.BlockSpec((BLK_R, EMB), lambda i: (i, 0)),
        ],
        out_shape=[
            jax.ShapeDtypeStruct((N, N), jnp.float32),
            jax.ShapeDtypeStruct((N, EMB), jnp.float32),
        ],
    )(h, h, WeT, be, wd, bd)


def kernel(nodes, edges, edge_attr, params):
    del nodes
    row, col = edges[0], edges[1]
    row2d = row.reshape(E // CHUNK, CHUNK)
    noise = jax.random.normal(jax.random.key(1), (N, 1), dtype=jnp.float32)

    g0 = params["gcl_0"]["edge_mlp_0"]
    A, B = _tc_prep0(noise, g0["W"][:, 0:1].T, g0["b"].reshape(1, H),
                     g0["W"][:, 1:2].T)
    h = noise
    col2d = col.reshape(E // CHUNK, CHUNK)
    for i in range(4):
        g = params["gcl_%d" % i]
        fin = 1 if i == 0 else H
        W0 = g["edge_mlp_0"]["W"]
        w0c = W0[:, 2 * fin].reshape(1, H)
        W1T = g["edge_mlp_1"]["W"].T
        b1 = g["edge_mlp_1"]["b"].reshape(1, H)

        t = _sc_gather_add(A, B, row2d, col2d)
        m = _tc_edge(t, edge_attr, w0c, W1T, b1)
        agg2 = _sc_scatter(m, row2d)

        Wn0 = g["node_mlp_0"]["W"]
        WhT = Wn0[:, :fin].T
        WaT = Wn0[:, fin:].T
        bn0 = g["node_mlp_0"]["b"].reshape(1, H)
        Wn1T = g["node_mlp_1"]["W"].T
        bn1 = g["node_mlp_1"]["b"].reshape(1, H)
        if i < 3:
            gn = params["gcl_%d" % (i + 1)]["edge_mlp_0"]
            h, A, B = _tc_node(h, agg2, WhT, WaT, bn0, Wn1T, bn1,
                               gn["W"][:, :H].T, gn["b"].reshape(1, H),
                               gn["W"][:, H:2 * H].T, first=(i == 0),
                               last=False)
        else:
            h = _tc_node(h, agg2, WhT, WaT, bn0, Wn1T, bn1,
                         None, None, None, first=False, last=True)

    fe, fd = params["fc_emb"], params["fc_dec"]
    adj, x = _tc_decode(h, fe["W"].T, fe["b"].reshape(1, EMB),
                        fd["W"].reshape(1, EMB), fd["b"].reshape(1, 1))
    return adj, x


# speedup vs baseline: 6.2995x; 1.0562x over previous
import functools

import jax
import jax.numpy as jnp
from jax import lax
from jax.experimental import pallas as pl
from jax.experimental.pallas import tpu as pltpu
from jax.experimental.pallas import tpu_sc as plsc

N = 2048
E = 65536
NSPLIT = 2
EH = E // NSPLIT
H = 128
EMB = 32
NW = 32
CHUNK = 128
NBUF = 3

_MESH = plsc.VectorSubcoreMesh(core_axis_name="c", subcore_axis_name="s")
_PREC = lax.Precision.HIGHEST


def _make_gather_add(ne):
    epw = ne // NW
    nchunk = epw // CHUNK
    scratch = (
        [pltpu.VMEM((nchunk, CHUNK), jnp.int32)] * 2
        + [pltpu.VMEM((CHUNK, H), jnp.float32)] * (2 * NBUF)
        + [pltpu.SemaphoreType.DMA] * (3 * NBUF)
    )

    @functools.partial(
        pl.kernel,
        out_type=jax.ShapeDtypeStruct((ne, H), jnp.float32),
        mesh=_MESH,
        scratch_types=scratch,
    )
    def gather_add(A_hbm, B_hbm, row_hbm, col_hbm, out_hbm, ir, ic, *bufs):
        bA = bufs[0:NBUF]
        bB = bufs[NBUF:2 * NBUF]
        sA = bufs[2 * NBUF:3 * NBUF]
        sB = bufs[3 * NBUF:4 * NBUF]
        sS = bufs[4 * NBUF:5 * NBUF]
        w = lax.axis_index("s") * 2 + lax.axis_index("c")
        pltpu.sync_copy(row_hbm.at[pl.ds(w * nchunk, nchunk), :], ir)
        pltpu.sync_copy(col_hbm.at[pl.ds(w * nchunk, nchunk), :], ic)

        def start_gather(j):
            p = j % NBUF
            return (pltpu.async_copy(A_hbm.at[ir.at[j]], bA[p], sA[p]),
                    pltpu.async_copy(B_hbm.at[ic.at[j]], bB[p], sB[p]))

        inflight = {0: start_gather(0)}
        if nchunk > 1:
            inflight[1] = start_gather(1)
        stores = {}
        for j in range(nchunk):
            p = j % NBUF
            ga, gb = inflight.pop(j)
            ga.wait()
            gb.wait()

            def addrow(r, carry, p=p):
                for l in range(H // 16):
                    bA[p][r, pl.ds(l * 16, 16)] += bB[p][r, pl.ds(l * 16, 16)]
                return carry

            lax.fori_loop(0, CHUNK, addrow, 0)
            stores[j] = pltpu.async_copy(
                bA[p], out_hbm.at[pl.ds(w * epw + j * CHUNK, CHUNK), :], sS[p])
            if j + 2 < nchunk:
                jn = j + 2
                if jn - NBUF in stores:
                    stores.pop(jn - NBUF).wait()
                inflight[jn] = start_gather(jn)
        for j in sorted(stores):
            stores.pop(j).wait()

    return gather_add


def _make_scatter(ne):
    epw = ne // NW
    nchunk = epw // CHUNK

    @functools.partial(
        pl.kernel,
        out_type=jax.ShapeDtypeStruct((2 * N, H), jnp.float32),
        mesh=_MESH,
        scratch_types=[
            pltpu.VMEM((nchunk, CHUNK), jnp.int32),
            pltpu.VMEM((CHUNK, H), jnp.float32),
            pltpu.VMEM((CHUNK, H), jnp.float32),
            pltpu.VMEM_SHARED((N, H), jnp.float32),
            pltpu.SemaphoreType.DMA,
            pltpu.SemaphoreType.DMA,
        ],
    )
    def scatter(m_hbm, idx_hbm, out_hbm, idx_v, mb0, mb1, acc_shared, s0, s1):
        c = lax.axis_index("c")
        s = lax.axis_index("s")
        w = s * 2 + c
        mb = (mb0, mb1)
        sm = (s0, s1)
        rpw = N // 16

        def zrow(r, carry):
            for l in range(H // 16):
                mb0[r, pl.ds(l * 16, 16)] = jnp.zeros((16,), jnp.float32)
            return carry

        lax.fori_loop(0, CHUNK, zrow, 0)
        pltpu.sync_copy(mb0, acc_shared.at[pl.ds(s * rpw, rpw), :])
        plsc.subcore_barrier()

        pltpu.sync_copy(idx_hbm.at[pl.ds(w * nchunk, nchunk), :], idx_v)

        def load(j):
            p = j % 2
            return pltpu.async_copy(
                m_hbm.at[pl.ds(w * epw + j * CHUNK, CHUNK), :], mb[p], sm[p])

        pend = {0: load(0)}
        for j in range(nchunk):
            p = j % 2
            pend.pop(j).wait()
            if j + 1 < nchunk:
                pend[j + 1] = load(j + 1)
            pltpu.sync_copy(mb[p], acc_shared.at[idx_v.at[j]], add=True)
        plsc.subcore_barrier()
        pltpu.sync_copy(acc_shared.at[pl.ds(s * rpw, rpw), :],
                        out_hbm.at[pl.ds(c * N + s * rpw, rpw), :])

    return scatter


_sc_gather_add = _make_gather_add(EH)
_sc_scatter = _make_scatter(EH)


BLK_E = 4096


def _edge_body(t, attr, w0c, W1T, b1, out):
    tt = t[...] + attr[...] * w0c[...]
    u = tt * jax.nn.sigmoid(tt)
    v = jnp.dot(u, W1T[...], preferred_element_type=jnp.float32,
                precision=_PREC) + b1[...]
    out[...] = v * jax.nn.sigmoid(v)


def _tc_edge(t, attr, w0c, W1T, b1):
    return pl.pallas_call(
        _edge_body,
        grid=(EH // BLK_E,),
        in_specs=[
            pl.BlockSpec((BLK_E, H), lambda i: (i, 0)),
            pl.BlockSpec((BLK_E, 1), lambda i: (i, 0)),
            pl.BlockSpec((1, H), lambda i: (0, 0)),
            pl.BlockSpec((H, H), lambda i: (0, 0)),
            pl.BlockSpec((1, H), lambda i: (0, 0)),
        ],
        out_specs=pl.BlockSpec((BLK_E, H), lambda i: (i, 0)),
        out_shape=jax.ShapeDtypeStruct((EH, H), jnp.float32),
    )(t, attr, w0c, W1T, b1)


def _node_body(h, agg2a, agg2b, WhT, WaT, bn0, Wn1T, bn1, WaTn, b0n, WbTn,
               h_out, A_out, B_out, *, first, last):
    ga = agg2a[...]
    gb = agg2b[...]
    agg = (ga[0:N] + ga[N:2 * N]) + (gb[0:N] + gb[N:2 * N])
    hh = h[...]
    if first:
        pre = hh * WhT[...] + bn0[...]
    else:
        pre = jnp.dot(hh, WhT[...], preferred_element_type=jnp.float32,
                      precision=_PREC) + bn0[...]
    pre = pre + jnp.dot(agg, WaT[...], preferred_element_type=jnp.float32,
                        precision=_PREC)
    u = pre * jax.nn.sigmoid(pre)
    hn = jnp.dot(u, Wn1T[...], preferred_element_type=jnp.float32,
                 precision=_PREC) + bn1[...]
    if not first:
        hn = hn + hh
    h_out[...] = hn
    if not last:
        A_out[...] = jnp.dot(hn, WaTn[...], preferred_element_type=jnp.float32,
                             precision=_PREC) + b0n[...]
        B_out[...] = jnp.dot(hn, WbTn[...], preferred_element_type=jnp.float32,
                             precision=_PREC)


def _tc_node(h, agg2a, agg2b, WhT, WaT, bn0, Wn1T, bn1, WaTn, b0n, WbTn,
             first, last):
    fin = 1 if first else H
    full = lambda shp: pl.BlockSpec(shp, lambda: tuple(0 for _ in shp))
    if last:
        body = functools.partial(_node_body, first=first, last=True)

        def body_last(h, agg2a, agg2b, WhT, WaT, bn0, Wn1T, bn1, h_out):
            body(h, agg2a, agg2b, WhT, WaT, bn0, Wn1T, bn1, None, None, None,
                 h_out, None, None)

        return pl.pallas_call(
            body_last,
            in_specs=[full((N, fin)), full((2 * N, H)), full((2 * N, H)),
                      full((fin, H)) if not first else full((1, H)),
                      full((H, H)), full((1, H)), full((H, H)), full((1, H))],
            out_specs=full((N, H)),
            out_shape=jax.ShapeDtypeStruct((N, H), jnp.float32),
        )(h, agg2a, agg2b, WhT, WaT, bn0, Wn1T, bn1)
    body = functools.partial(_node_body, first=first, last=False)
    return pl.pallas_call(
        body,
        in_specs=[full((N, fin)), full((2 * N, H)), full((2 * N, H)),
                  full((fin, H)) if not first else full((1, H)),
                  full((H, H)), full((1, H)), full((H, H)), full((1, H)),
                  full((H, H)), full((1, H)), full((H, H))],
        out_specs=[full((N, H))] * 3,
        out_shape=[jax.ShapeDtypeStruct((N, H), jnp.float32)] * 3,
    )(h, agg2a, agg2b, WhT, WaT, bn0, Wn1T, bn1, WaTn, b0n, WbTn)


def _prep0_body(noise, wa, b0, wb, A_out, B_out):
    nz = noise[...]
    A_out[...] = nz * wa[...] + b0[...]
    B_out[...] = nz * wb[...]


def _tc_prep0(noise, wa, b0, wb):
    full = lambda shp: pl.BlockSpec(shp, lambda: tuple(0 for _ in shp))
    return pl.pallas_call(
        _prep0_body,
        in_specs=[full((N, 1)), full((1, H)), full((1, H)), full((1, H))],
        out_specs=[full((N, H))] * 2,
        out_shape=[jax.ShapeDtypeStruct((N, H), jnp.float32)] * 2,
    )(noise, wa, b0, wb)


BLK_R = 256


def _dec_body(hb, hf, WeT, be, wd, bd, adj_out, x_out):
    i = pl.program_id(0)
    xb = jnp.dot(hb[...], WeT[...], preferred_element_type=jnp.float32,
                 precision=_PREC) + be[...]
    xf = jnp.dot(hf[...], WeT[...], preferred_element_type=jnp.float32,
                 precision=_PREC) + be[...]
    qb = xb * wd[...]
    sb = jnp.sum(qb * xb, axis=1, keepdims=True)
    srow = lax.dot_general(wd[...], xf * xf, (((1,), (1,)), ((), ())),
                           preferred_element_type=jnp.float32,
                           precision=_PREC)
    G = lax.dot_general(qb, xf, (((1,), (1,)), ((), ())),
                        preferred_element_type=jnp.float32, precision=_PREC)
    z = sb + srow - 2.0 * G + bd[...]
    a = jax.nn.sigmoid(z)
    rid = lax.broadcasted_iota(jnp.int32, (BLK_R, N), 0) + i * BLK_R
    cid = lax.broadcasted_iota(jnp.int32, (BLK_R, N), 1)
    adj_out[...] = jnp.where(rid == cid, 0.0, a)
    x_out[...] = xb


def _tc_decode(h, WeT, be, wd, bd):
    return pl.pallas_call(
        _dec_body,
        grid=(N // BLK_R,),
        in_specs=[
            pl.BlockSpec((BLK_R, H), lambda i: (i, 0)),
            pl.BlockSpec((N, H), lambda i: (0, 0)),
            pl.BlockSpec((H, EMB), lambda i: (0, 0)),
            pl.BlockSpec((1, EMB), lambda i: (0, 0)),
            pl.BlockSpec((1, EMB), lambda i: (0, 0)),
            pl.BlockSpec((1, 1), lambda i: (0, 0)),
        ],
        out_specs=[
            pl.BlockSpec((BLK_R, N), lambda i: (i, 0)),
            pl.BlockSpec((BLK_R, EMB), lambda i: (i, 0)),
        ],
        out_shape=[
            jax.ShapeDtypeStruct((N, N), jnp.float32),
            jax.ShapeDtypeStruct((N, EMB), jnp.float32),
        ],
    )(h, h, WeT, be, wd, bd)


def kernel(nodes, edges, edge_attr, params):
    del nodes
    row, col = edges[0], edges[1]
    rows2d = [row[k * EH:(k + 1) * EH].reshape(EH // CHUNK, CHUNK)
              for k in range(NSPLIT)]
    cols2d = [col[k * EH:(k + 1) * EH].reshape(EH // CHUNK, CHUNK)
              for k in range(NSPLIT)]
    attrs = [edge_attr[k * EH:(k + 1) * EH] for k in range(NSPLIT)]
    noise = jax.random.normal(jax.random.key(1), (N, 1), dtype=jnp.float32)

    g0 = params["gcl_0"]["edge_mlp_0"]
    A, B = _tc_prep0(noise, g0["W"][:, 0:1].T, g0["b"].reshape(1, H),
                     g0["W"][:, 1:2].T)
    h = noise
    for i in range(4):
        g = params["gcl_%d" % i]
        fin = 1 if i == 0 else H
        W0 = g["edge_mlp_0"]["W"]
        w0c = W0[:, 2 * fin].reshape(1, H)
        W1T = g["edge_mlp_1"]["W"].T
        b1 = g["edge_mlp_1"]["b"].reshape(1, H)

        ts = [_sc_gather_add(A, B, rows2d[k], cols2d[k])
              for k in range(NSPLIT)]
        ms = [_tc_edge(ts[k], attrs[k], w0c, W1T, b1) for k in range(NSPLIT)]
        aggs = [_sc_scatter(ms[k], rows2d[k]) for k in range(NSPLIT)]

        Wn0 = g["node_mlp_0"]["W"]
        WhT = Wn0[:, :fin].T
        WaT = Wn0[:, fin:].T
        bn0 = g["node_mlp_0"]["b"].reshape(1, H)
        Wn1T = g["node_mlp_1"]["W"].T
        bn1 = g["node_mlp_1"]["b"].reshape(1, H)
        if i < 3:
            gn = params["gcl_%d" % (i + 1)]["edge_mlp_0"]
            h, A, B = _tc_node(h, aggs[0], aggs[1], WhT, WaT, bn0, Wn1T, bn1,
                               gn["W"][:, :H].T, gn["b"].reshape(1, H),
                               gn["W"][:, H:2 * H].T, first=(i == 0),
                               last=False)
        else:
            h = _tc_node(h, aggs[0], aggs[1], WhT, WaT, bn0, Wn1T, bn1,
                         None, None, None, first=False, last=True)

    fe, fd = params["fc_emb"], params["fc_dec"]
    adj, x = _tc_decode(h, fe["W"].T, fe["b"].reshape(1, EMB),
                        fd["W"].reshape(1, EMB), fd["b"].reshape(1, 1))
    return adj, x


# speedup vs baseline: 6.3256x; 1.0042x over previous
import functools

import jax
import jax.numpy as jnp
from jax import lax
from jax.experimental import pallas as pl
from jax.experimental.pallas import tpu as pltpu
from jax.experimental.pallas import tpu_sc as plsc

N = 2048
E = 65536
NSPLIT = 2
EH = E // NSPLIT
H = 128
EMB = 32
NW = 32
CHUNK = 128
NBUF = 3

_MESH = plsc.VectorSubcoreMesh(core_axis_name="c", subcore_axis_name="s")
_PREC = lax.Precision.HIGHEST


def _make_gather_add(ne):
    epw = ne // NW
    nchunk = epw // CHUNK
    scratch = (
        [pltpu.VMEM((2 * nchunk, CHUNK), jnp.int32)]
        + [pltpu.VMEM((CHUNK, H), jnp.float32)] * (2 * NBUF)
        + [pltpu.SemaphoreType.DMA] * (3 * NBUF)
    )

    @functools.partial(
        pl.kernel,
        out_type=jax.ShapeDtypeStruct((ne, H), jnp.float32),
        mesh=_MESH,
        scratch_types=scratch,
    )
    def gather_add(A_hbm, B_hbm, rc_hbm, out_hbm, irc, *bufs):
        bA = bufs[0:NBUF]
        bB = bufs[NBUF:2 * NBUF]
        sA = bufs[2 * NBUF:3 * NBUF]
        sB = bufs[3 * NBUF:4 * NBUF]
        sS = bufs[4 * NBUF:5 * NBUF]
        w = lax.axis_index("s") * 2 + lax.axis_index("c")
        pltpu.sync_copy(rc_hbm.at[pl.ds(w * 2 * nchunk, 2 * nchunk), :], irc)

        def start_gather(j):
            p = j % NBUF
            return (pltpu.async_copy(A_hbm.at[irc.at[j]], bA[p], sA[p]),
                    pltpu.async_copy(B_hbm.at[irc.at[nchunk + j]], bB[p],
                                     sB[p]))

        inflight = {0: start_gather(0)}
        if nchunk > 1:
            inflight[1] = start_gather(1)
        stores = {}
        for j in range(nchunk):
            p = j % NBUF
            ga, gb = inflight.pop(j)
            ga.wait()
            gb.wait()

            def addrow(r, carry, p=p):
                for l in range(H // 16):
                    bA[p][r, pl.ds(l * 16, 16)] += bB[p][r, pl.ds(l * 16, 16)]
                return carry

            lax.fori_loop(0, CHUNK, addrow, 0)
            stores[j] = pltpu.async_copy(
                bA[p], out_hbm.at[pl.ds(w * epw + j * CHUNK, CHUNK), :], sS[p])
            if j + 2 < nchunk:
                jn = j + 2
                if jn - NBUF in stores:
                    stores.pop(jn - NBUF).wait()
                inflight[jn] = start_gather(jn)
        for j in sorted(stores):
            stores.pop(j).wait()

    return gather_add


def _make_scatter(ne):
    epw = ne // NW
    nchunk = epw // CHUNK

    @functools.partial(
        pl.kernel,
        out_type=jax.ShapeDtypeStruct((2 * N, H), jnp.float32),
        mesh=_MESH,
        scratch_types=[
            pltpu.VMEM((nchunk, CHUNK), jnp.int32),
            pltpu.VMEM((CHUNK, H), jnp.float32),
            pltpu.VMEM((CHUNK, H), jnp.float32),
            pltpu.VMEM_SHARED((N, H), jnp.float32),
            pltpu.SemaphoreType.DMA,
            pltpu.SemaphoreType.DMA,
        ],
    )
    def scatter(m_hbm, idx_hbm, out_hbm, idx_v, mb0, mb1, acc_shared, s0, s1):
        c = lax.axis_index("c")
        s = lax.axis_index("s")
        w = s * 2 + c
        mb = (mb0, mb1)
        sm = (s0, s1)
        rpw = N // 16

        def zrow(r, carry):
            for l in range(H // 16):
                mb0[r, pl.ds(l * 16, 16)] = jnp.zeros((16,), jnp.float32)
            return carry

        lax.fori_loop(0, CHUNK, zrow, 0)
        pltpu.sync_copy(mb0, acc_shared.at[pl.ds(s * rpw, rpw), :])
        plsc.subcore_barrier()

        pltpu.sync_copy(idx_hbm.at[pl.ds(w * nchunk, nchunk), :], idx_v)

        def load(j):
            p = j % 2
            return pltpu.async_copy(
                m_hbm.at[pl.ds(w * epw + j * CHUNK, CHUNK), :], mb[p], sm[p])

        pend = {0: load(0)}
        for j in range(nchunk):
            p = j % 2
            pend.pop(j).wait()
            if j + 1 < nchunk:
                pend[j + 1] = load(j + 1)
            pltpu.sync_copy(mb[p], acc_shared.at[idx_v.at[j]], add=True)
        plsc.subcore_barrier()
        pltpu.sync_copy(acc_shared.at[pl.ds(s * rpw, rpw), :],
                        out_hbm.at[pl.ds(c * N + s * rpw, rpw), :])

    return scatter


_sc_gather_add = _make_gather_add(EH)
_sc_scatter = _make_scatter(EH)


BLK_E = 4096


def _edge_body(t, attr, w0c, W1T, b1, out):
    tt = t[...] + attr[...] * w0c[...]
    u = tt * jax.nn.sigmoid(tt)
    v = jnp.dot(u, W1T[...], preferred_element_type=jnp.float32) + b1[...]
    out[...] = v * jax.nn.sigmoid(v)


def _tc_edge(t, attr, w0c, W1T, b1):
    return pl.pallas_call(
        _edge_body,
        grid=(EH // BLK_E,),
        in_specs=[
            pl.BlockSpec((BLK_E, H), lambda i: (i, 0)),
            pl.BlockSpec((BLK_E, 1), lambda i: (i, 0)),
            pl.BlockSpec((1, H), lambda i: (0, 0)),
            pl.BlockSpec((H, H), lambda i: (0, 0)),
            pl.BlockSpec((1, H), lambda i: (0, 0)),
        ],
        out_specs=pl.BlockSpec((BLK_E, H), lambda i: (i, 0)),
        out_shape=jax.ShapeDtypeStruct((EH, H), jnp.float32),
    )(t, attr, w0c, W1T, b1)


def _node_body(h, agg2a, agg2b, WhT, WaT, bn0, Wn1T, bn1, WaTn, b0n, WbTn,
               h_out, A_out, B_out, *, first, last):
    ga = agg2a[...]
    gb = agg2b[...]
    agg = (ga[0:N] + ga[N:2 * N]) + (gb[0:N] + gb[N:2 * N])
    hh = h[...]
    if first:
        pre = hh * WhT[...] + bn0[...]
    else:
        pre = jnp.dot(hh, WhT[...], preferred_element_type=jnp.float32,
                      precision=_PREC) + bn0[...]
    pre = pre + jnp.dot(agg, WaT[...], preferred_element_type=jnp.float32,
                        precision=_PREC)
    u = pre * jax.nn.sigmoid(pre)
    hn = jnp.dot(u, Wn1T[...], preferred_element_type=jnp.float32,
                 precision=_PREC) + bn1[...]
    if not first:
        hn = hn + hh
    h_out[...] = hn
    if not last:
        A_out[...] = jnp.dot(hn, WaTn[...], preferred_element_type=jnp.float32,
                             precision=_PREC) + b0n[...]
        B_out[...] = jnp.dot(hn, WbTn[...], preferred_element_type=jnp.float32,
                             precision=_PREC)


def _tc_node(h, agg2a, agg2b, WhT, WaT, bn0, Wn1T, bn1, WaTn, b0n, WbTn,
             first, last):
    fin = 1 if first else H
    full = lambda shp: pl.BlockSpec(shp, lambda: tuple(0 for _ in shp))
    if last:
        body = functools.partial(_node_body, first=first, last=True)

        def body_last(h, agg2a, agg2b, WhT, WaT, bn0, Wn1T, bn1, h_out):
            body(h, agg2a, agg2b, WhT, WaT, bn0, Wn1T, bn1, None, None, None,
                 h_out, None, None)

        return pl.pallas_call(
            body_last,
            in_specs=[full((N, fin)), full((2 * N, H)), full((2 * N, H)),
                      full((fin, H)) if not first else full((1, H)),
                      full((H, H)), full((1, H)), full((H, H)), full((1, H))],
            out_specs=full((N, H)),
            out_shape=jax.ShapeDtypeStruct((N, H), jnp.float32),
        )(h, agg2a, agg2b, WhT, WaT, bn0, Wn1T, bn1)
    body = functools.partial(_node_body, first=first, last=False)
    return pl.pallas_call(
        body,
        in_specs=[full((N, fin)), full((2 * N, H)), full((2 * N, H)),
                  full((fin, H)) if not first else full((1, H)),
                  full((H, H)), full((1, H)), full((H, H)), full((1, H)),
                  full((H, H)), full((1, H)), full((H, H))],
        out_specs=[full((N, H))] * 3,
        out_shape=[jax.ShapeDtypeStruct((N, H), jnp.float32)] * 3,
    )(h, agg2a, agg2b, WhT, WaT, bn0, Wn1T, bn1, WaTn, b0n, WbTn)


def _prep0_body(noise, wa, b0, wb, A_out, B_out):
    nz = noise[...]
    A_out[...] = nz * wa[...] + b0[...]
    B_out[...] = nz * wb[...]


def _tc_prep0(noise, wa, b0, wb):
    full = lambda shp: pl.BlockSpec(shp, lambda: tuple(0 for _ in shp))
    return pl.pallas_call(
        _prep0_body,
        in_specs=[full((N, 1)), full((1, H)), full((1, H)), full((1, H))],
        out_specs=[full((N, H))] * 2,
        out_shape=[jax.ShapeDtypeStruct((N, H), jnp.float32)] * 2,
    )(noise, wa, b0, wb)


BLK_R = 256


def _dec_body(hb, hf, WeT, be, wd, bd, adj_out, x_out):
    i = pl.program_id(0)
    xb = jnp.dot(hb[...], WeT[...], preferred_element_type=jnp.float32,
                 precision=_PREC) + be[...]
    xf = jnp.dot(hf[...], WeT[...], preferred_element_type=jnp.float32,
                 precision=_PREC) + be[...]
    qb = xb * wd[...]
    sb = jnp.sum(qb * xb, axis=1, keepdims=True)
    srow = lax.dot_general(wd[...], xf * xf, (((1,), (1,)), ((), ())),
                           preferred_element_type=jnp.float32,
                           precision=_PREC)
    G = lax.dot_general(qb, xf, (((1,), (1,)), ((), ())),
                        preferred_element_type=jnp.float32, precision=_PREC)
    z = sb + srow - 2.0 * G + bd[...]
    a = jax.nn.sigmoid(z)
    rid = lax.broadcasted_iota(jnp.int32, (BLK_R, N), 0) + i * BLK_R
    cid = lax.broadcasted_iota(jnp.int32, (BLK_R, N), 1)
    adj_out[...] = jnp.where(rid == cid, 0.0, a)
    x_out[...] = xb


def _tc_decode(h, WeT, be, wd, bd):
    return pl.pallas_call(
        _dec_body,
        grid=(N // BLK_R,),
        in_specs=[
            pl.BlockSpec((BLK_R, H), lambda i: (i, 0)),
            pl.BlockSpec((N, H), lambda i: (0, 0)),
            pl.BlockSpec((H, EMB), lambda i: (0, 0)),
            pl.BlockSpec((1, EMB), lambda i: (0, 0)),
            pl.BlockSpec((1, EMB), lambda i: (0, 0)),
            pl.BlockSpec((1, 1), lambda i: (0, 0)),
        ],
        out_specs=[
            pl.BlockSpec((BLK_R, N), lambda i: (i, 0)),
            pl.BlockSpec((BLK_R, EMB), lambda i: (i, 0)),
        ],
        out_shape=[
            jax.ShapeDtypeStruct((N, N), jnp.float32),
            jax.ShapeDtypeStruct((N, EMB), jnp.float32),
        ],
    )(h, h, WeT, be, wd, bd)


def kernel(nodes, edges, edge_attr, params):
    del nodes
    row, col = edges[0], edges[1]
    nch = EH // NW // CHUNK
    rows2d = [row[k * EH:(k + 1) * EH].reshape(EH // CHUNK, CHUNK)
              for k in range(NSPLIT)]
    rcs2d = [jnp.concatenate(
        [row[k * EH:(k + 1) * EH].reshape(NW, 1, nch, CHUNK),
         col[k * EH:(k + 1) * EH].reshape(NW, 1, nch, CHUNK)],
        axis=1).reshape(2 * EH // CHUNK, CHUNK) for k in range(NSPLIT)]
    attrs = [edge_attr[k * EH:(k + 1) * EH] for k in range(NSPLIT)]
    noise = jax.random.normal(jax.random.key(1), (N, 1), dtype=jnp.float32)

    g0 = params["gcl_0"]["edge_mlp_0"]
    A, B = _tc_prep0(noise, g0["W"][:, 0:1].T, g0["b"].reshape(1, H),
                     g0["W"][:, 1:2].T)
    h = noise
    for i in range(4):
        g = params["gcl_%d" % i]
        fin = 1 if i == 0 else H
        W0 = g["edge_mlp_0"]["W"]
        w0c = W0[:, 2 * fin].reshape(1, H)
        W1T = g["edge_mlp_1"]["W"].T
        b1 = g["edge_mlp_1"]["b"].reshape(1, H)

        ts = [_sc_gather_add(A, B, rcs2d[k]) for k in range(NSPLIT)]
        ms = [_tc_edge(ts[k], attrs[k], w0c, W1T, b1) for k in range(NSPLIT)]
        aggs = [_sc_scatter(ms[k], rows2d[k]) for k in range(NSPLIT)]

        Wn0 = g["node_mlp_0"]["W"]
        WhT = Wn0[:, :fin].T
        WaT = Wn0[:, fin:].T
        bn0 = g["node_mlp_0"]["b"].reshape(1, H)
        Wn1T = g["node_mlp_1"]["W"].T
        bn1 = g["node_mlp_1"]["b"].reshape(1, H)
        if i < 3:
            gn = params["gcl_%d" % (i + 1)]["edge_mlp_0"]
            h, A, B = _tc_node(h, aggs[0], aggs[1], WhT, WaT, bn0, Wn1T, bn1,
                               gn["W"][:, :H].T, gn["b"].reshape(1, H),
                               gn["W"][:, H:2 * H].T, first=(i == 0),
                               last=False)
        else:
            h = _tc_node(h, aggs[0], aggs[1], WhT, WaT, bn0, Wn1T, bn1,
                         None, None, None, first=False, last=True)

    fe, fd = params["fc_emb"], params["fc_dec"]
    adj, x = _tc_decode(h, fe["W"].T, fe["b"].reshape(1, EMB),
                        fd["W"].reshape(1, EMB), fd["b"].reshape(1, 1))
    return adj, x


# speedup vs baseline: 6.3948x; 1.0109x over previous
import functools

import jax
import jax.numpy as jnp
from jax import lax
from jax.experimental import pallas as pl
from jax.experimental.pallas import tpu as pltpu
from jax.experimental.pallas import tpu_sc as plsc

N = 2048
E = 65536
NSPLIT = 2
EH = E // NSPLIT
H = 128
EMB = 32
NW = 32
CHUNK = 128
NBUF = 3

_MESH = plsc.VectorSubcoreMesh(core_axis_name="c", subcore_axis_name="s")
_PREC = lax.Precision.HIGHEST


def _make_gather_add(ne):
    epw = ne // NW
    nchunk = epw // CHUNK
    scratch = (
        [pltpu.VMEM((2 * nchunk, CHUNK), jnp.int32)]
        + [pltpu.VMEM((CHUNK, H), jnp.float32)] * (2 * NBUF)
        + [pltpu.SemaphoreType.DMA] * (3 * NBUF)
    )

    @functools.partial(
        pl.kernel,
        out_type=jax.ShapeDtypeStruct((ne, H), jnp.float32),
        mesh=_MESH,
        scratch_types=scratch,
        compiler_params=pltpu.CompilerParams(use_tc_tiling_on_sc=True),
    )
    def gather_add(A_hbm, B_hbm, rc_hbm, out_hbm, irc, *bufs):
        bA = bufs[0:NBUF]
        bB = bufs[NBUF:2 * NBUF]
        sA = bufs[2 * NBUF:3 * NBUF]
        sB = bufs[3 * NBUF:4 * NBUF]
        sS = bufs[4 * NBUF:5 * NBUF]
        w = lax.axis_index("s") * 2 + lax.axis_index("c")
        pltpu.sync_copy(rc_hbm.at[pl.ds(w * 2 * nchunk, 2 * nchunk), :], irc)

        def start_gather(j):
            p = j % NBUF
            return (pltpu.async_copy(A_hbm.at[irc.at[j]], bA[p], sA[p]),
                    pltpu.async_copy(B_hbm.at[irc.at[nchunk + j]], bB[p],
                                     sB[p]))

        inflight = {0: start_gather(0)}
        if nchunk > 1:
            inflight[1] = start_gather(1)
        stores = {}
        for j in range(nchunk):
            p = j % NBUF
            ga, gb = inflight.pop(j)
            ga.wait()
            gb.wait()
            if j + 2 < nchunk:
                jn = j + 2
                if jn - NBUF in stores:
                    stores.pop(jn - NBUF).wait()
                inflight[jn] = start_gather(jn)

            def addrow(r, carry, p=p):
                for l in range(H // 16):
                    bA[p][r, pl.ds(l * 16, 16)] += bB[p][r, pl.ds(l * 16, 16)]
                return carry

            lax.fori_loop(0, CHUNK, addrow, 0)
            stores[j] = pltpu.async_copy(
                bA[p], out_hbm.at[pl.ds(w * epw + j * CHUNK, CHUNK), :], sS[p])
        for j in sorted(stores):
            stores.pop(j).wait()

    return gather_add


def _make_scatter(ne):
    epw = ne // NW
    nchunk = epw // CHUNK

    @functools.partial(
        pl.kernel,
        out_type=jax.ShapeDtypeStruct((2 * N, H), jnp.float32),
        mesh=_MESH,
        scratch_types=[
            pltpu.VMEM((nchunk, CHUNK), jnp.int32),
            pltpu.VMEM((CHUNK, H), jnp.float32),
            pltpu.VMEM((CHUNK, H), jnp.float32),
            pltpu.VMEM_SHARED((N, H), jnp.float32),
            pltpu.SemaphoreType.DMA,
            pltpu.SemaphoreType.DMA,
        ],
        compiler_params=pltpu.CompilerParams(use_tc_tiling_on_sc=True),
    )
    def scatter(m_hbm, idx_hbm, out_hbm, idx_v, mb0, mb1, acc_shared, s0, s1):
        c = lax.axis_index("c")
        s = lax.axis_index("s")
        w = s * 2 + c
        mb = (mb0, mb1)
        sm = (s0, s1)
        rpw = N // 16

        def zrow(r, carry):
            for l in range(H // 16):
                mb0[r, pl.ds(l * 16, 16)] = jnp.zeros((16,), jnp.float32)
            return carry

        lax.fori_loop(0, CHUNK, zrow, 0)
        pltpu.sync_copy(mb0, acc_shared.at[pl.ds(s * rpw, rpw), :])
        plsc.subcore_barrier()

        pltpu.sync_copy(idx_hbm.at[pl.ds(w * nchunk, nchunk), :], idx_v)

        def load(j):
            p = j % 2
            return pltpu.async_copy(
                m_hbm.at[pl.ds(w * epw + j * CHUNK, CHUNK), :], mb[p], sm[p])

        pend = {0: load(0)}
        for j in range(nchunk):
            p = j % 2
            pend.pop(j).wait()
            if j + 1 < nchunk:
                pend[j + 1] = load(j + 1)
            pltpu.sync_copy(mb[p], acc_shared.at[idx_v.at[j]], add=True)
        plsc.subcore_barrier()
        pltpu.sync_copy(acc_shared.at[pl.ds(s * rpw, rpw), :],
                        out_hbm.at[pl.ds(c * N + s * rpw, rpw), :])

    return scatter


_sc_gather_add = _make_gather_add(EH)
_sc_scatter = _make_scatter(EH)


BLK_E = 4096


def _edge_body(t, attr, w0c, W1T, b1, out):
    tt = t[...] + attr[...] * w0c[...]
    u = tt * jax.nn.sigmoid(tt)
    v = jnp.dot(u, W1T[...], preferred_element_type=jnp.float32) + b1[...]
    out[...] = v * jax.nn.sigmoid(v)


def _tc_edge(t, attr, w0c, W1T, b1):
    return pl.pallas_call(
        _edge_body,
        grid=(EH // BLK_E,),
        in_specs=[
            pl.BlockSpec((BLK_E, H), lambda i: (i, 0)),
            pl.BlockSpec((BLK_E, 1), lambda i: (i, 0)),
            pl.BlockSpec((1, H), lambda i: (0, 0)),
            pl.BlockSpec((H, H), lambda i: (0, 0)),
            pl.BlockSpec((1, H), lambda i: (0, 0)),
        ],
        out_specs=pl.BlockSpec((BLK_E, H), lambda i: (i, 0)),
        out_shape=jax.ShapeDtypeStruct((EH, H), jnp.float32),
    )(t, attr, w0c, W1T, b1)


def _node_body(h, agg2a, agg2b, WhT, WaT, bn0, Wn1T, bn1, WaTn, b0n, WbTn,
               h_out, A_out, B_out, *, first, last):
    ga = agg2a[...]
    gb = agg2b[...]
    agg = (ga[0:N] + ga[N:2 * N]) + (gb[0:N] + gb[N:2 * N])
    hh = h[...]
    if first:
        pre = hh * WhT[...] + bn0[...]
    else:
        pre = jnp.dot(hh, WhT[...], preferred_element_type=jnp.float32,
                      precision=_PREC) + bn0[...]
    pre = pre + jnp.dot(agg, WaT[...], preferred_element_type=jnp.float32,
                        precision=_PREC)
    u = pre * jax.nn.sigmoid(pre)
    hn = jnp.dot(u, Wn1T[...], preferred_element_type=jnp.float32,
                 precision=_PREC) + bn1[...]
    if not first:
        hn = hn + hh
    h_out[...] = hn
    if not last:
        A_out[...] = jnp.dot(hn, WaTn[...], preferred_element_type=jnp.float32,
                             precision=_PREC) + b0n[...]
        B_out[...] = jnp.dot(hn, WbTn[...], preferred_element_type=jnp.float32,
                             precision=_PREC)


def _tc_node(h, agg2a, agg2b, WhT, WaT, bn0, Wn1T, bn1, WaTn, b0n, WbTn,
             first, last):
    fin = 1 if first else H
    full = lambda shp: pl.BlockSpec(shp, lambda: tuple(0 for _ in shp))
    if last:
        body = functools.partial(_node_body, first=first, last=True)

        def body_last(h, agg2a, agg2b, WhT, WaT, bn0, Wn1T, bn1, h_out):
            body(h, agg2a, agg2b, WhT, WaT, bn0, Wn1T, bn1, None, None, None,
                 h_out, None, None)

        return pl.pallas_call(
            body_last,
            in_specs=[full((N, fin)), full((2 * N, H)), full((2 * N, H)),
                      full((fin, H)) if not first else full((1, H)),
                      full((H, H)), full((1, H)), full((H, H)), full((1, H))],
            out_specs=full((N, H)),
            out_shape=jax.ShapeDtypeStruct((N, H), jnp.float32),
        )(h, agg2a, agg2b, WhT, WaT, bn0, Wn1T, bn1)
    body = functools.partial(_node_body, first=first, last=False)
    return pl.pallas_call(
        body,
        in_specs=[full((N, fin)), full((2 * N, H)), full((2 * N, H)),
                  full((fin, H)) if not first else full((1, H)),
                  full((H, H)), full((1, H)), full((H, H)), full((1, H)),
                  full((H, H)), full((1, H)), full((H, H))],
        out_specs=[full((N, H))] * 3,
        out_shape=[jax.ShapeDtypeStruct((N, H), jnp.float32)] * 3,
    )(h, agg2a, agg2b, WhT, WaT, bn0, Wn1T, bn1, WaTn, b0n, WbTn)


def _prep0_body(noise, wa, b0, wb, A_out, B_out):
    nz = noise[...]
    A_out[...] = nz * wa[...] + b0[...]
    B_out[...] = nz * wb[...]


def _tc_prep0(noise, wa, b0, wb):
    full = lambda shp: pl.BlockSpec(shp, lambda: tuple(0 for _ in shp))
    return pl.pallas_call(
        _prep0_body,
        in_specs=[full((N, 1)), full((1, H)), full((1, H)), full((1, H))],
        out_specs=[full((N, H))] * 2,
        out_shape=[jax.ShapeDtypeStruct((N, H), jnp.float32)] * 2,
    )(noise, wa, b0, wb)


BLK_R = 256


def _dec_body(hb, hf, WeT, be, wd, bd, adj_out, x_out):
    i = pl.program_id(0)
    xb = jnp.dot(hb[...], WeT[...], preferred_element_type=jnp.float32,
                 precision=_PREC) + be[...]
    xf = jnp.dot(hf[...], WeT[...], preferred_element_type=jnp.float32,
                 precision=_PREC) + be[...]
    qb = xb * wd[...]
    sb = jnp.sum(qb * xb, axis=1, keepdims=True)
    srow = lax.dot_general(wd[...], xf * xf, (((1,), (1,)), ((), ())),
                           preferred_element_type=jnp.float32,
                           precision=_PREC)
    G = lax.dot_general(qb, xf, (((1,), (1,)), ((), ())),
                        preferred_element_type=jnp.float32, precision=_PREC)
    z = sb + srow - 2.0 * G + bd[...]
    a = jax.nn.sigmoid(z)
    rid = lax.broadcasted_iota(jnp.int32, (BLK_R, N), 0) + i * BLK_R
    cid = lax.broadcasted_iota(jnp.int32, (BLK_R, N), 1)
    adj_out[...] = jnp.where(rid == cid, 0.0, a)
    x_out[...] = xb


def _tc_decode(h, WeT, be, wd, bd):
    return pl.pallas_call(
        _dec_body,
        grid=(N // BLK_R,),
        in_specs=[
            pl.BlockSpec((BLK_R, H), lambda i: (i, 0)),
            pl.BlockSpec((N, H), lambda i: (0, 0)),
            pl.BlockSpec((H, EMB), lambda i: (0, 0)),
            pl.BlockSpec((1, EMB), lambda i: (0, 0)),
            pl.BlockSpec((1, EMB), lambda i: (0, 0)),
            pl.BlockSpec((1, 1), lambda i: (0, 0)),
        ],
        out_specs=[
            pl.BlockSpec((BLK_R, N), lambda i: (i, 0)),
            pl.BlockSpec((BLK_R, EMB), lambda i: (i, 0)),
        ],
        out_shape=[
            jax.ShapeDtypeStruct((N, N), jnp.float32),
            jax.ShapeDtypeStruct((N, EMB), jnp.float32),
        ],
    )(h, h, WeT, be, wd, bd)


def kernel(nodes, edges, edge_attr, params):
    del nodes
    row, col = edges[0], edges[1]
    nch = EH // NW // CHUNK
    rows2d = [row[k * EH:(k + 1) * EH].reshape(EH // CHUNK, CHUNK)
              for k in range(NSPLIT)]
    rcs2d = [jnp.concatenate(
        [row[k * EH:(k + 1) * EH].reshape(NW, 1, nch, CHUNK),
         col[k * EH:(k + 1) * EH].reshape(NW, 1, nch, CHUNK)],
        axis=1).reshape(2 * EH // CHUNK, CHUNK) for k in range(NSPLIT)]
    attrs = [edge_attr[k * EH:(k + 1) * EH] for k in range(NSPLIT)]
    noise = jax.random.normal(jax.random.key(1), (N, 1), dtype=jnp.float32)

    g0 = params["gcl_0"]["edge_mlp_0"]
    A, B = _tc_prep0(noise, g0["W"][:, 0:1].T, g0["b"].reshape(1, H),
                     g0["W"][:, 1:2].T)
    h = noise
    for i in range(4):
        g = params["gcl_%d" % i]
        fin = 1 if i == 0 else H
        W0 = g["edge_mlp_0"]["W"]
        w0c = W0[:, 2 * fin].reshape(1, H)
        W1T = g["edge_mlp_1"]["W"].T
        b1 = g["edge_mlp_1"]["b"].reshape(1, H)

        ts = [_sc_gather_add(A, B, rcs2d[k]) for k in range(NSPLIT)]
        ms = [_tc_edge(ts[k], attrs[k], w0c, W1T, b1) for k in range(NSPLIT)]
        aggs = [_sc_scatter(ms[k], rows2d[k]) for k in range(NSPLIT)]

        Wn0 = g["node_mlp_0"]["W"]
        WhT = Wn0[:, :fin].T
        WaT = Wn0[:, fin:].T
        bn0 = g["node_mlp_0"]["b"].reshape(1, H)
        Wn1T = g["node_mlp_1"]["W"].T
        bn1 = g["node_mlp_1"]["b"].reshape(1, H)
        if i < 3:
            gn = params["gcl_%d" % (i + 1)]["edge_mlp_0"]
            h, A, B = _tc_node(h, aggs[0], aggs[1], WhT, WaT, bn0, Wn1T, bn1,
                               gn["W"][:, :H].T, gn["b"].reshape(1, H),
                               gn["W"][:, H:2 * H].T, first=(i == 0),
                               last=False)
        else:
            h = _tc_node(h, aggs[0], aggs[1], WhT, WaT, bn0, Wn1T, bn1,
                         None, None, None, first=False, last=True)

    fe, fd = params["fc_emb"], params["fc_dec"]
    adj, x = _tc_decode(h, fe["W"].T, fe["b"].reshape(1, EMB),
                        fd["W"].reshape(1, EMB), fd["b"].reshape(1, 1))
    return adj, x
